# baseline (speedup 1.0000x reference)
import functools

import jax
import jax.numpy as jnp
from jax import lax
from jax.experimental import pallas as pl
from jax.experimental.pallas import tpu as pltpu
from jax.experimental.pallas import tpu_sc as plsc

VOCAB = 1000000
EMB = 64
NCLASS = 9
BATCH = 16384
WIN = 5

NC = 2
NS = 16
NW = NC * NS

TOTAL_ROWS = BATCH * WIN
ROWS_PER_W = TOTAL_ROWS // NW
CHUNK = 128
NCHUNK = ROWS_PER_W // CHUNK


def _sc_gather_body(table_hbm, idx_hbm, out_hbm, idx_v, rows_a, rows_b, sem_a, sem_b):
    wid = lax.axis_index("s") * NC + lax.axis_index("c")
    base = wid * ROWS_PER_W
    pltpu.sync_copy(idx_hbm.at[wid], idx_v)

    pltpu.async_copy(table_hbm.at[idx_v.at[0]], rows_a, sem_a)

    def step(j, carry):
        del carry
        cp1 = pltpu.async_copy(table_hbm.at[idx_v.at[j + 1]], rows_b, sem_b)
        pltpu.make_async_copy(table_hbm.at[idx_v.at[0]], rows_a, sem_a).wait()
        pltpu.sync_copy(rows_a, out_hbm.at[pl.ds(base + j * CHUNK, CHUNK)])

        @pl.when(j + 2 < NCHUNK)
        def _():
            pltpu.async_copy(table_hbm.at[idx_v.at[j + 2]], rows_a, sem_a)

        cp1.wait()
        pltpu.sync_copy(rows_b, out_hbm.at[pl.ds(base + (j + 1) * CHUNK, CHUNK)])
        return 0

    lax.fori_loop(0, NCHUNK // 2, lambda s, c: step(2 * s, c), 0)


def _sc_gather(table, idx3):
    k = pl.kernel(
        _sc_gather_body,
        out_type=jax.ShapeDtypeStruct((TOTAL_ROWS, EMB), jnp.float32),
        mesh=plsc.VectorSubcoreMesh(core_axis_name="c", subcore_axis_name="s"),
        scratch_types=[
            pltpu.VMEM((NCHUNK, CHUNK), jnp.int32),
            pltpu.VMEM((CHUNK, EMB), jnp.float32),
            pltpu.VMEM((CHUNK, EMB), jnp.float32),
            pltpu.SemaphoreType.DMA,
            pltpu.SemaphoreType.DMA,
        ],
        compiler_params=pltpu.CompilerParams(use_tc_tiling_on_sc=False),
    )
    return k(table, idx3)


def _tc_matmul_body(x_ref, v_ref, b_ref, o_ref):
    w = pl.program_id(1)

    @pl.when(w == 0)
    def _():
        o_ref[...] = jnp.broadcast_to(b_ref[...], o_ref.shape)

    o_ref[...] += jnp.dot(
        x_ref[...], v_ref[0], preferred_element_type=jnp.float32
    )


def _tc_matmul(rows, v, b2):
    blk = 2048
    nb = BATCH // blk
    return pl.pallas_call(
        _tc_matmul_body,
        grid=(nb, WIN),
        in_specs=[
            pl.BlockSpec((blk, EMB), lambda i, w: (w * nb + i, 0)),
            pl.BlockSpec((1, EMB, NCLASS), lambda i, w: (w, 0, 0)),
            pl.BlockSpec((1, NCLASS), lambda i, w: (0, 0)),
        ],
        out_specs=pl.BlockSpec((blk, NCLASS), lambda i, w: (i, 0)),
        out_shape=jax.ShapeDtypeStruct((BATCH, NCLASS), jnp.float32),
    )(rows, v, b2)


@jax.jit
def kernel(x, table, W, b):
    idx3 = x.T.reshape(NW, NCHUNK, CHUNK)
    rows = _sc_gather(table, idx3)
    v = W.reshape(NCLASS, WIN, EMB).transpose(1, 2, 0)
    return _tc_matmul(rows, v, b.reshape(1, NCLASS))


# speedup vs baseline: 1.0363x; 1.0363x over previous
import functools

import jax
import jax.numpy as jnp
from jax import lax
from jax.experimental import pallas as pl
from jax.experimental.pallas import tpu as pltpu
from jax.experimental.pallas import tpu_sc as plsc

VOCAB = 1000000
EMB = 64
NCLASS = 9
BATCH = 16384
WIN = 5

NC = 2
NS = 16
NW = NC * NS

TOTAL_ROWS = BATCH * WIN
ROWS_PER_W = TOTAL_ROWS // NW
CHUNK = 128
NCHUNK = ROWS_PER_W // CHUNK
HALF = CHUNK // 2
LINES = TOTAL_ROWS // 2
LINES_PER_W = ROWS_PER_W // 2
LINES_PER_CHUNK = HALF


def _sc_gather_body(table_hbm, idx_hbm, out_hbm,
                    idx_v, buf_ea, buf_oa, buf_eb, buf_ob,
                    sem_ea, sem_oa, sem_eb, sem_ob):
    wid = lax.axis_index("s") * NC + lax.axis_index("c")
    lbase = wid * LINES_PER_W
    pltpu.sync_copy(idx_hbm.at[wid], idx_v)

    def fire(j, buf_e, buf_o, sem_e, sem_o):
        pltpu.async_copy(
            table_hbm.at[idx_v.at[j, pl.ds(0, HALF)]], buf_e, sem_e)
        pltpu.async_copy(
            table_hbm.at[idx_v.at[j, pl.ds(HALF, HALF)]], buf_o, sem_o)

    def store(j, buf_e, buf_o, sem_e, sem_o):
        l0 = lbase + j * LINES_PER_CHUNK
        pltpu.make_async_copy(
            table_hbm.at[idx_v.at[0, pl.ds(0, HALF)]], buf_e, sem_e).wait()
        pltpu.sync_copy(
            buf_e, out_hbm.at[pl.ds(l0, LINES_PER_CHUNK), pl.ds(0, EMB)])
        pltpu.make_async_copy(
            table_hbm.at[idx_v.at[0, pl.ds(0, HALF)]], buf_o, sem_o).wait()
        pltpu.sync_copy(
            buf_o, out_hbm.at[pl.ds(l0, LINES_PER_CHUNK), pl.ds(EMB, EMB)])

    fire(0, buf_ea, buf_oa, sem_ea, sem_oa)

    def step(j, carry):
        del carry
        fire(j + 1, buf_eb, buf_ob, sem_eb, sem_ob)
        store(j, buf_ea, buf_oa, sem_ea, sem_oa)

        @pl.when(j + 2 < NCHUNK)
        def _():
            fire(j + 2, buf_ea, buf_oa, sem_ea, sem_oa)

        store(j + 1, buf_eb, buf_ob, sem_eb, sem_ob)
        return 0

    lax.fori_loop(0, NCHUNK // 2, lambda s, c: step(2 * s, c), 0)


def _sc_gather(table, idx3):
    k = pl.kernel(
        _sc_gather_body,
        out_type=jax.ShapeDtypeStruct((LINES, 2 * EMB), jnp.float32),
        mesh=plsc.VectorSubcoreMesh(core_axis_name="c", subcore_axis_name="s"),
        scratch_types=[
            pltpu.VMEM((NCHUNK, CHUNK), jnp.int32),
            pltpu.VMEM((HALF, EMB), jnp.float32),
            pltpu.VMEM((HALF, EMB), jnp.float32),
            pltpu.VMEM((HALF, EMB), jnp.float32),
            pltpu.VMEM((HALF, EMB), jnp.float32),
            pltpu.SemaphoreType.DMA,
            pltpu.SemaphoreType.DMA,
            pltpu.SemaphoreType.DMA,
            pltpu.SemaphoreType.DMA,
        ],
        compiler_params=pltpu.CompilerParams(use_tc_tiling_on_sc=False),
    )
    return k(table, idx3)


def _tc_matmul_body(x_ref, u_ref, b_ref, o_ref):
    w = pl.program_id(1)

    @pl.when(w == 0)
    def _():
        o_ref[...] = jnp.broadcast_to(b_ref[...], o_ref.shape)

    o_ref[...] += jnp.dot(
        x_ref[...], u_ref[0], preferred_element_type=jnp.float32
    )


def _tc_matmul(lines, u, b2):
    blk = 1024
    nb = (BATCH // 2) // blk
    return pl.pallas_call(
        _tc_matmul_body,
        grid=(nb, WIN),
        in_specs=[
            pl.BlockSpec((blk, 2 * EMB), lambda i, w: (w * nb + i, 0)),
            pl.BlockSpec((1, 2 * EMB, 2 * NCLASS), lambda i, w: (w, 0, 0)),
            pl.BlockSpec((1, 2 * NCLASS), lambda i, w: (0, 0)),
        ],
        out_specs=pl.BlockSpec((blk, 2 * NCLASS), lambda i, w: (i, 0)),
        out_shape=jax.ShapeDtypeStruct((BATCH // 2, 2 * NCLASS), jnp.float32),
    )(lines, u, b2)


@jax.jit
def kernel(x, table, W, b):
    f = x.T.reshape(NW, NCHUNK, HALF, 2)
    idx3 = f.transpose(0, 1, 3, 2).reshape(NW, NCHUNK, CHUNK)
    lines = _sc_gather(table, idx3)

    v = W.reshape(NCLASS, WIN, EMB).transpose(1, 2, 0)
    u = jnp.zeros((WIN, 2 * EMB, 2 * NCLASS), jnp.float32)
    u = u.at[:, :EMB, :NCLASS].set(v).at[:, EMB:, NCLASS:].set(v)
    b2 = jnp.concatenate([b, b]).reshape(1, 2 * NCLASS)

    z = _tc_matmul(lines, u, b2)
    return z.reshape(BATCH, NCLASS)


# speedup vs baseline: 1.2557x; 1.2116x over previous
import functools

import jax
import jax.numpy as jnp
from jax import lax
from jax.experimental import pallas as pl
from jax.experimental.pallas import tpu as pltpu
from jax.experimental.pallas import tpu_sc as plsc

VOCAB = 1000000
EMB = 64
NCLASS = 9
BATCH = 16384
WIN = 5

NC = 2
NS = 16
NW = NC * NS

TOTAL_ROWS = BATCH * WIN
ROWS_PER_W = TOTAL_ROWS // NW
CHUNK = 128
NCHUNK = ROWS_PER_W // CHUNK
HALF = CHUNK // 2
LINES = TOTAL_ROWS // 2
LINES_PER_W = ROWS_PER_W // 2
LINES_PER_CHUNK = HALF


def _sc_gather_body(table_hbm, idx_hbm, out_hbm,
                    idx_v, buf_ea, buf_oa, buf_eb, buf_ob,
                    sem_ea, sem_oa, sem_eb, sem_ob):
    wid = lax.axis_index("s") * NC + lax.axis_index("c")
    lbase = wid * LINES_PER_W
    pltpu.sync_copy(idx_hbm.at[wid], idx_v)

    def fire(j, buf_e, buf_o, sem_e, sem_o):
        pltpu.async_copy(
            table_hbm.at[idx_v.at[j, pl.ds(0, HALF)]], buf_e, sem_e)
        pltpu.async_copy(
            table_hbm.at[idx_v.at[j, pl.ds(HALF, HALF)]], buf_o, sem_o)

    def store(j, buf_e, buf_o, sem_e, sem_o):
        l0 = lbase + j * LINES_PER_CHUNK
        pltpu.make_async_copy(
            table_hbm.at[idx_v.at[0, pl.ds(0, HALF)]], buf_e, sem_e).wait()
        pltpu.sync_copy(
            buf_e, out_hbm.at[pl.ds(l0, LINES_PER_CHUNK), pl.ds(0, EMB)])
        pltpu.make_async_copy(
            table_hbm.at[idx_v.at[0, pl.ds(0, HALF)]], buf_o, sem_o).wait()
        pltpu.sync_copy(
            buf_o, out_hbm.at[pl.ds(l0, LINES_PER_CHUNK), pl.ds(EMB, EMB)])

    fire(0, buf_ea, buf_oa, sem_ea, sem_oa)

    def step(j, carry):
        del carry
        fire(j + 1, buf_eb, buf_ob, sem_eb, sem_ob)
        store(j, buf_ea, buf_oa, sem_ea, sem_oa)

        @pl.when(j + 2 < NCHUNK)
        def _():
            fire(j + 2, buf_ea, buf_oa, sem_ea, sem_oa)

        store(j + 1, buf_eb, buf_ob, sem_eb, sem_ob)
        return 0

    lax.fori_loop(0, NCHUNK // 2, lambda s, c: step(2 * s, c), 0)


def _sc_gather(table, idx3):
    k = pl.kernel(
        _sc_gather_body,
        out_type=jax.ShapeDtypeStruct((LINES, 2 * EMB), jnp.float32),
        mesh=plsc.VectorSubcoreMesh(core_axis_name="c", subcore_axis_name="s"),
        scratch_types=[
            pltpu.VMEM((NCHUNK, CHUNK), jnp.int32),
            pltpu.VMEM((HALF, EMB), jnp.float32),
            pltpu.VMEM((HALF, EMB), jnp.float32),
            pltpu.VMEM((HALF, EMB), jnp.float32),
            pltpu.VMEM((HALF, EMB), jnp.float32),
            pltpu.SemaphoreType.DMA,
            pltpu.SemaphoreType.DMA,
            pltpu.SemaphoreType.DMA,
            pltpu.SemaphoreType.DMA,
        ],
        compiler_params=pltpu.CompilerParams(use_tc_tiling_on_sc=False),
    )
    return k(table, idx3)


TBLK = 2048
TLINES = TBLK // 2
TGRID = -(-VOCAB // TBLK)


def _tc_transpose_body(t_ref, o_ref):
    tt = t_ref[...].T
    r = tt.reshape(TBLK // 128, 2, 64, EMB)
    left = r[:, 0].reshape(TLINES, EMB)
    right = r[:, 1].reshape(TLINES, EMB)
    o_ref[...] = jnp.concatenate([left, right], axis=-1)


def _tc_transpose(table_t):
    return pl.pallas_call(
        _tc_transpose_body,
        grid=(TGRID,),
        in_specs=[pl.BlockSpec((EMB, TBLK), lambda c: (0, c))],
        out_specs=pl.BlockSpec((TLINES, 2 * EMB), lambda c: (c, 0)),
        out_shape=jax.ShapeDtypeStruct((TGRID * TLINES, 2 * EMB), jnp.float32),
    )(table_t)


def _tc_matmul_body(x_ref, u_ref, b_ref, o_ref):
    w = pl.program_id(1)

    @pl.when(w == 0)
    def _():
        o_ref[...] = jnp.broadcast_to(b_ref[...], o_ref.shape)

    o_ref[...] += jnp.dot(
        x_ref[...], u_ref[0], preferred_element_type=jnp.float32
    )


def _tc_matmul(lines, u, b2):
    blk = 1024
    nb = (BATCH // 2) // blk
    return pl.pallas_call(
        _tc_matmul_body,
        grid=(nb, WIN),
        in_specs=[
            pl.BlockSpec((blk, 2 * EMB), lambda i, w: (w * nb + i, 0)),
            pl.BlockSpec((1, 2 * EMB, 2 * NCLASS), lambda i, w: (w, 0, 0)),
            pl.BlockSpec((1, 2 * NCLASS), lambda i, w: (0, 0)),
        ],
        out_specs=pl.BlockSpec((blk, 2 * NCLASS), lambda i, w: (i, 0)),
        out_shape=jax.ShapeDtypeStruct((BATCH // 2, 2 * NCLASS), jnp.float32),
    )(lines, u, b2)


@jax.jit
def kernel(x, table, W, b):
    packed = _tc_transpose(table.T)
    table_rm = packed.reshape(2 * TGRID * TLINES, EMB)
    xm = (x // 128) * 128 + 2 * (x % 64) + ((x % 128) // 64)

    f = xm.T.reshape(NW, NCHUNK, HALF, 2)
    idx3 = f.transpose(0, 1, 3, 2).reshape(NW, NCHUNK, CHUNK)
    lines = _sc_gather(table_rm, idx3)

    v = W.reshape(NCLASS, WIN, EMB).transpose(1, 2, 0)
    u = jnp.zeros((WIN, 2 * EMB, 2 * NCLASS), jnp.float32)
    u = u.at[:, :EMB, :NCLASS].set(v).at[:, EMB:, NCLASS:].set(v)
    b2 = jnp.concatenate([b, b]).reshape(1, 2 * NCLASS)

    z = _tc_matmul(lines, u, b2)
    return z.reshape(BATCH, NCLASS)


# speedup vs baseline: 2.0755x; 1.6529x over previous
import functools

import jax
import jax.numpy as jnp
from jax import lax
from jax.experimental import pallas as pl
from jax.experimental.pallas import tpu as pltpu
from jax.experimental.pallas import tpu_sc as plsc

VOCAB = 1000000
EMB = 64
NCLASS = 9
BATCH = 16384
WIN = 5

NC = 2
NS = 16
NW = NC * NS

TOTAL_ROWS = BATCH * WIN
ROWS_PER_W = TOTAL_ROWS // NW
CHUNK = 128
NCHUNK = ROWS_PER_W // CHUNK
HALF = CHUNK // 2
LINES = TOTAL_ROWS // 2
LINES_PER_W = ROWS_PER_W // 2
LINES_PER_CHUNK = HALF


def _sc_gather_body(table_hbm, idx_hbm, out_hbm,
                    idx_v, buf_ea, buf_oa, buf_eb, buf_ob,
                    sem_ea, sem_oa, sem_eb, sem_ob):
    wid = lax.axis_index("s") * NC + lax.axis_index("c")
    lbase = wid * LINES_PER_W
    pltpu.sync_copy(idx_hbm.at[wid], idx_v)

    def fire(j, buf_e, buf_o, sem_e, sem_o):
        pltpu.async_copy(
            table_hbm.at[idx_v.at[j, pl.ds(0, HALF)]], buf_e, sem_e)
        pltpu.async_copy(
            table_hbm.at[idx_v.at[j, pl.ds(HALF, HALF)]], buf_o, sem_o)

    def store(j, buf_e, buf_o, sem_e, sem_o):
        l0 = lbase + j * LINES_PER_CHUNK
        pltpu.make_async_copy(
            table_hbm.at[idx_v.at[0, pl.ds(0, HALF)]], buf_e, sem_e).wait()
        pltpu.sync_copy(
            buf_e, out_hbm.at[pl.ds(l0, LINES_PER_CHUNK), pl.ds(0, EMB)])
        pltpu.make_async_copy(
            table_hbm.at[idx_v.at[0, pl.ds(0, HALF)]], buf_o, sem_o).wait()
        pltpu.sync_copy(
            buf_o, out_hbm.at[pl.ds(l0, LINES_PER_CHUNK), pl.ds(EMB, EMB)])

    fire(0, buf_ea, buf_oa, sem_ea, sem_oa)

    def step(j, carry):
        del carry
        fire(j + 1, buf_eb, buf_ob, sem_eb, sem_ob)
        store(j, buf_ea, buf_oa, sem_ea, sem_oa)

        @pl.when(j + 2 < NCHUNK)
        def _():
            fire(j + 2, buf_ea, buf_oa, sem_ea, sem_oa)

        store(j + 1, buf_eb, buf_ob, sem_eb, sem_ob)
        return 0

    lax.fori_loop(0, NCHUNK // 2, lambda s, c: step(2 * s, c), 0)


def _sc_gather(table, idx3):
    k = pl.kernel(
        _sc_gather_body,
        out_type=jax.ShapeDtypeStruct((LINES, 2 * EMB), jnp.float32),
        mesh=plsc.VectorSubcoreMesh(core_axis_name="c", subcore_axis_name="s"),
        scratch_types=[
            pltpu.VMEM((NCHUNK, CHUNK), jnp.int32),
            pltpu.VMEM((HALF, EMB), jnp.float32),
            pltpu.VMEM((HALF, EMB), jnp.float32),
            pltpu.VMEM((HALF, EMB), jnp.float32),
            pltpu.VMEM((HALF, EMB), jnp.float32),
            pltpu.SemaphoreType.DMA,
            pltpu.SemaphoreType.DMA,
            pltpu.SemaphoreType.DMA,
            pltpu.SemaphoreType.DMA,
        ],
        compiler_params=pltpu.CompilerParams(use_tc_tiling_on_sc=False),
    )
    return k(table, idx3)


TBLK = 16384
TLINES = TBLK // 2
TGRID = -(-VOCAB // TBLK)


def _tc_transpose_body(t_ref, o_ref):
    tt = t_ref[...].T
    r = tt.reshape(TBLK // 128, 2, 64, EMB)
    left = r[:, 0].reshape(TLINES, EMB)
    right = r[:, 1].reshape(TLINES, EMB)
    o_ref[...] = jnp.concatenate([left, right], axis=-1)


def _tc_transpose(table_t):
    return pl.pallas_call(
        _tc_transpose_body,
        grid=(TGRID,),
        in_specs=[pl.BlockSpec((EMB, TBLK), lambda c: (0, c))],
        out_specs=pl.BlockSpec((TLINES, 2 * EMB), lambda c: (c, 0)),
        out_shape=jax.ShapeDtypeStruct((TGRID * TLINES, 2 * EMB), jnp.float32),
    )(table_t)


def _tc_matmul_body(x_ref, u_ref, b_ref, o_ref):
    w = pl.program_id(1)

    @pl.when(w == 0)
    def _():
        o_ref[...] = jnp.broadcast_to(b_ref[...], o_ref.shape)

    o_ref[...] += jnp.dot(
        x_ref[...], u_ref[0], preferred_element_type=jnp.float32
    )


def _tc_matmul(lines, u, b2):
    blk = 1024
    nb = (BATCH // 2) // blk
    return pl.pallas_call(
        _tc_matmul_body,
        grid=(nb, WIN),
        in_specs=[
            pl.BlockSpec((blk, 2 * EMB), lambda i, w: (w * nb + i, 0)),
            pl.BlockSpec((1, 2 * EMB, 2 * NCLASS), lambda i, w: (w, 0, 0)),
            pl.BlockSpec((1, 2 * NCLASS), lambda i, w: (0, 0)),
        ],
        out_specs=pl.BlockSpec((blk, 2 * NCLASS), lambda i, w: (i, 0)),
        out_shape=jax.ShapeDtypeStruct((BATCH // 2, 2 * NCLASS), jnp.float32),
    )(lines, u, b2)


@jax.jit
def kernel(x, table, W, b):
    packed = _tc_transpose(table.T)
    table_rm = packed.reshape(2 * TGRID * TLINES, EMB)
    xm = (x // 128) * 128 + 2 * (x % 64) + ((x % 128) // 64)

    f = xm.T.reshape(NW, NCHUNK, HALF, 2)
    idx3 = f.transpose(0, 1, 3, 2).reshape(NW, NCHUNK, CHUNK)
    lines = _sc_gather(table_rm, idx3)

    v = W.reshape(NCLASS, WIN, EMB).transpose(1, 2, 0)
    u = jnp.zeros((WIN, 2 * EMB, 2 * NCLASS), jnp.float32)
    u = u.at[:, :EMB, :NCLASS].set(v).at[:, EMB:, NCLASS:].set(v)
    b2 = jnp.concatenate([b, b]).reshape(1, 2 * NCLASS)

    z = _tc_matmul(lines, u, b2)
    return z.reshape(BATCH, NCLASS)


# speedup vs baseline: 2.1626x; 1.0420x over previous
import functools

import jax
import jax.numpy as jnp
from jax import lax
from jax.experimental import pallas as pl
from jax.experimental.pallas import tpu as pltpu
from jax.experimental.pallas import tpu_sc as plsc

VOCAB = 1000000
EMB = 64
NCLASS = 9
BATCH = 16384
WIN = 5

NC = 2
NS = 16
NW = NC * NS

TOTAL_ROWS = BATCH * WIN
ROWS_PER_W = TOTAL_ROWS // NW
CHUNK = 128
NCHUNK = ROWS_PER_W // CHUNK
HALF = CHUNK // 2
LINES = TOTAL_ROWS // 2
LINES_PER_W = ROWS_PER_W // 2
LINES_PER_CHUNK = HALF


def _sc_gather_body(table_hbm, idx_hbm, out_hbm,
                    idx_v, buf_ea, buf_oa, buf_eb, buf_ob,
                    sem_ea, sem_oa, sem_eb, sem_ob):
    wid = lax.axis_index("s") * NC + lax.axis_index("c")
    lbase = wid * LINES_PER_W
    pltpu.sync_copy(idx_hbm.at[wid], idx_v)

    def fire(j, buf_e, buf_o, sem_e, sem_o):
        pltpu.async_copy(
            table_hbm.at[idx_v.at[j, pl.ds(0, HALF)]], buf_e, sem_e)
        pltpu.async_copy(
            table_hbm.at[idx_v.at[j, pl.ds(HALF, HALF)]], buf_o, sem_o)

    def store(j, buf_e, buf_o, sem_e, sem_o):
        l0 = lbase + j * LINES_PER_CHUNK
        pltpu.make_async_copy(
            table_hbm.at[idx_v.at[0, pl.ds(0, HALF)]], buf_e, sem_e).wait()
        pltpu.sync_copy(
            buf_e, out_hbm.at[pl.ds(l0, LINES_PER_CHUNK), pl.ds(0, EMB)])
        pltpu.make_async_copy(
            table_hbm.at[idx_v.at[0, pl.ds(0, HALF)]], buf_o, sem_o).wait()
        pltpu.sync_copy(
            buf_o, out_hbm.at[pl.ds(l0, LINES_PER_CHUNK), pl.ds(EMB, EMB)])

    fire(0, buf_ea, buf_oa, sem_ea, sem_oa)

    def step(j, carry):
        del carry
        fire(j + 1, buf_eb, buf_ob, sem_eb, sem_ob)
        store(j, buf_ea, buf_oa, sem_ea, sem_oa)

        @pl.when(j + 2 < NCHUNK)
        def _():
            fire(j + 2, buf_ea, buf_oa, sem_ea, sem_oa)

        store(j + 1, buf_eb, buf_ob, sem_eb, sem_ob)
        return 0

    lax.fori_loop(0, NCHUNK // 2, lambda s, c: step(2 * s, c), 0)


def _sc_gather(table, idx3):
    k = pl.kernel(
        _sc_gather_body,
        out_type=jax.ShapeDtypeStruct((LINES, 2 * EMB), jnp.float32),
        mesh=plsc.VectorSubcoreMesh(core_axis_name="c", subcore_axis_name="s"),
        scratch_types=[
            pltpu.VMEM((NCHUNK, CHUNK), jnp.int32),
            pltpu.VMEM((HALF, EMB), jnp.float32),
            pltpu.VMEM((HALF, EMB), jnp.float32),
            pltpu.VMEM((HALF, EMB), jnp.float32),
            pltpu.VMEM((HALF, EMB), jnp.float32),
            pltpu.SemaphoreType.DMA,
            pltpu.SemaphoreType.DMA,
            pltpu.SemaphoreType.DMA,
            pltpu.SemaphoreType.DMA,
        ],
        compiler_params=pltpu.CompilerParams(use_tc_tiling_on_sc=False),
    )
    return k(table, idx3)


TBLK = 32768
TLINES = TBLK // 2
TGRID = -(-VOCAB // TBLK)


def _tc_transpose_body(t_ref, o_ref):
    tt = t_ref[...].T
    r = tt.reshape(TBLK // 128, 2, 64, EMB)
    left = r[:, 0].reshape(TLINES, EMB)
    right = r[:, 1].reshape(TLINES, EMB)
    o_ref[...] = jnp.concatenate([left, right], axis=-1)


def _tc_transpose(table_t):
    return pl.pallas_call(
        _tc_transpose_body,
        grid=(TGRID,),
        in_specs=[pl.BlockSpec((EMB, TBLK), lambda c: (0, c))],
        out_specs=pl.BlockSpec((TLINES, 2 * EMB), lambda c: (c, 0)),
        out_shape=jax.ShapeDtypeStruct((TGRID * TLINES, 2 * EMB), jnp.float32),
    )(table_t)


def _tc_matmul_body(x_ref, u_ref, b_ref, o_ref):
    w = pl.program_id(1)

    @pl.when(w == 0)
    def _():
        o_ref[...] = jnp.broadcast_to(b_ref[...], o_ref.shape)

    o_ref[...] += jnp.dot(
        x_ref[...], u_ref[0], preferred_element_type=jnp.float32
    )


def _tc_matmul(lines, u, b2):
    blk = 1024
    nb = (BATCH // 2) // blk
    return pl.pallas_call(
        _tc_matmul_body,
        grid=(nb, WIN),
        in_specs=[
            pl.BlockSpec((blk, 2 * EMB), lambda i, w: (w * nb + i, 0)),
            pl.BlockSpec((1, 2 * EMB, 2 * NCLASS), lambda i, w: (w, 0, 0)),
            pl.BlockSpec((1, 2 * NCLASS), lambda i, w: (0, 0)),
        ],
        out_specs=pl.BlockSpec((blk, 2 * NCLASS), lambda i, w: (i, 0)),
        out_shape=jax.ShapeDtypeStruct((BATCH // 2, 2 * NCLASS), jnp.float32),
    )(lines, u, b2)


@jax.jit
def kernel(x, table, W, b):
    packed = _tc_transpose(table.T)
    table_rm = packed.reshape(2 * TGRID * TLINES, EMB)
    xm = (x // 128) * 128 + 2 * (x % 64) + ((x % 128) // 64)

    f = xm.T.reshape(NW, NCHUNK, HALF, 2)
    idx3 = f.transpose(0, 1, 3, 2).reshape(NW, NCHUNK, CHUNK)
    lines = _sc_gather(table_rm, idx3)

    v = W.reshape(NCLASS, WIN, EMB).transpose(1, 2, 0)
    u = jnp.zeros((WIN, 2 * EMB, 2 * NCLASS), jnp.float32)
    u = u.at[:, :EMB, :NCLASS].set(v).at[:, EMB:, NCLASS:].set(v)
    b2 = jnp.concatenate([b, b]).reshape(1, 2 * NCLASS)

    z = _tc_matmul(lines, u, b2)
    return z.reshape(BATCH, NCLASS)
